# baseline (speedup 1.0000x reference)
import functools

import jax
import jax.numpy as jnp
from jax import lax
from jax.experimental import pallas as pl
from jax.experimental.pallas import tpu as pltpu
from jax.experimental.pallas import tpu_sc as plsc

N = 10000
E = 320000
C = 128
STEPS = 3
NT = 3
NG = 16
LEAKY = 0.1

R = 30208
HR = R // 2
TRASH = 3 * N
EIR = 5120
NSUB = 16
PTR = EIR // NSUB
GRP = 4
RPT = HR // NSUB
BM = 1000
NB = N // BM


def _mk_f32(shape):
    return jax.ShapeDtypeStruct(shape, jnp.float32)


def _build_h(x_type2d, x_tok2d, x_small):
    def body(xt_ref, xk_ref, xs_ref, h_ref):
        col = lax.broadcasted_iota(jnp.int32, (BM, C), 1)
        xt = xt_ref[...]
        xk = jnp.clip(xk_ref[...], 0, 61)
        xs = xs_ref[...]
        one = jnp.float32(1.0)
        zero = jnp.float32(0.0)
        h = jnp.where(col == xt, one, zero)
        h = h + jnp.where(col - 64 == xk, one, zero)
        h = h + jnp.where(col == 126, xs[:, 0:1], zero)
        h = h + jnp.where(col == 127, xs[:, 1:2], zero)
        h_ref[...] = h

    return pl.pallas_call(
        body,
        grid=(NB,),
        in_specs=[
            pl.BlockSpec((BM, 1), lambda i: (i, 0)),
            pl.BlockSpec((BM, 1), lambda i: (i, 0)),
            pl.BlockSpec((BM, 2), lambda i: (i, 0)),
        ],
        out_specs=pl.BlockSpec((BM, C), lambda i: (i, 0)),
        out_shape=_mk_f32((N, C)),
    )(x_type2d, x_tok2d, x_small)


def _edge_indices(src2d, dst2d, et2d):
    def body(s_ref, d_ref, t_ref, g_ref, o_ref):
        t = t_ref[...]
        g_ref[...] = t * N + s_ref[...]
        o_ref[...] = t * N + d_ref[...]

    blk = pl.BlockSpec((2500, 128), lambda: (0, 0))
    return pl.pallas_call(
        body,
        grid=(),
        in_specs=[blk, blk, blk],
        out_specs=[blk, blk],
        out_shape=[jax.ShapeDtypeStruct((2500, 128), jnp.int32)] * 2,
    )(src2d, dst2d, et2d)


def _mm3(x, w3, x_is_h):
    def body(x_ref, w_ref, o_ref):
        o_ref[...] = lax.dot_general(
            x_ref[0], w_ref[0], (((1,), (0,)), ((), ())),
            preferred_element_type=jnp.float32)

    if x_is_h:
        x = x.reshape(1, N, C)
        x_map = lambda t, i: (0, i, 0)
    else:
        x_map = lambda t, i: (t, i, 0)
    return pl.pallas_call(
        body,
        grid=(NT, NB),
        in_specs=[
            pl.BlockSpec((1, BM, C), x_map),
            pl.BlockSpec((1, C, C), lambda t, i: (t, 0, 0)),
        ],
        out_specs=pl.BlockSpec((BM, C), lambda t, i: (t * NB + i, 0)),
        out_shape=_mk_f32((R, C)),
    )(x, w3)


def _gru3(msg, x, wih3, whh3, bih3, bhh3, x_is_h):
    def body(m_ref, x_ref, wi_ref, wh_ref, bi_ref, bh_ref, o_ref):
        m = m_ref[...]
        h = x_ref[0]
        gi = lax.dot_general(m, wi_ref[0], (((1,), (1,)), ((), ())),
                             preferred_element_type=jnp.float32) + bi_ref[0]
        gh = lax.dot_general(h, wh_ref[0], (((1,), (1,)), ((), ())),
                             preferred_element_type=jnp.float32) + bh_ref[0]
        r = jax.nn.sigmoid(gi[:, :C] + gh[:, :C])
        z = jax.nn.sigmoid(gi[:, C:2 * C] + gh[:, C:2 * C])
        n = jnp.tanh(gi[:, 2 * C:] + r * gh[:, 2 * C:])
        o_ref[0] = (1.0 - z) * n + z * h

    if x_is_h:
        x = x.reshape(1, N, C)
        x_map = lambda t, i: (0, i, 0)
    else:
        x_map = lambda t, i: (t, i, 0)
    return pl.pallas_call(
        body,
        grid=(NT, NB),
        in_specs=[
            pl.BlockSpec((BM, C), lambda t, i: (t * NB + i, 0)),
            pl.BlockSpec((1, BM, C), x_map),
            pl.BlockSpec((1, 3 * C, C), lambda t, i: (t, 0, 0)),
            pl.BlockSpec((1, 3 * C, C), lambda t, i: (t, 0, 0)),
            pl.BlockSpec((1, 1, 3 * C), lambda t, i: (t, 0, 0)),
            pl.BlockSpec((1, 1, 3 * C), lambda t, i: (t, 0, 0)),
        ],
        out_specs=pl.BlockSpec((1, BM, C), lambda t, i: (t, i, 0)),
        out_shape=_mk_f32((NT, N, C)),
    )(msg, x, wih3, whh3, bih3, bhh3)


def _block_end(h, xs, ln_g, ln_b):
    def body(h_ref, xs_ref, g_ref, b_ref, o_ref):
        tot = h_ref[...] + xs_ref[0] + xs_ref[1] + xs_ref[2]
        mu = jnp.mean(tot, axis=-1, keepdims=True)
        d = tot - mu
        var = jnp.mean(d * d, axis=-1, keepdims=True)
        y = d * lax.rsqrt(var + 1e-5) * g_ref[...] + b_ref[...]
        o_ref[...] = jnp.where(y >= 0, y, LEAKY * y)

    return pl.pallas_call(
        body,
        grid=(NB,),
        in_specs=[
            pl.BlockSpec((BM, C), lambda i: (i, 0)),
            pl.BlockSpec((3, BM, C), lambda i: (0, i, 0)),
            pl.BlockSpec((1, C), lambda i: (0, 0)),
            pl.BlockSpec((1, C), lambda i: (0, 0)),
        ],
        out_specs=pl.BlockSpec((BM, C), lambda i: (i, 0)),
        out_shape=_mk_f32((N, C)),
    )(h, xs, ln_g, ln_b)


def _pool_head(h, batch2d, gw1, gb1, gw2, gb2, hw1, hb1, hw2, hb2):
    def body(h_ref, b_ref, gw1_ref, gb1_ref, gw2_ref, gb2_ref,
             hw1_ref, hb1_ref, hw2_ref, hb2_ref, o_ref):
        h = h_ref[...]
        g1 = lax.dot_general(h, gw1_ref[...], (((1,), (1,)), ((), ())),
                             preferred_element_type=jnp.float32) + gb1_ref[...]
        g1 = jnp.where(g1 >= 0, g1, LEAKY * g1)
        g = jnp.sum(g1 * gw2_ref[...], axis=1, keepdims=True) + gb2_ref[0, 0]
        b = b_ref[...]
        oh = (b == lax.broadcasted_iota(jnp.int32, (N, NG), 1)).astype(jnp.float32)
        neg = jnp.float32(-3e38)
        gmax = jnp.max(jnp.where(oh > 0, g, neg), axis=0, keepdims=True)
        gmax = jnp.where(gmax > -1e38, gmax, 0.0)
        gmax_b = jnp.sum(oh * gmax, axis=1, keepdims=True)
        ge = jnp.exp(g - gmax_b)
        gs = jnp.sum(oh * ge, axis=0, keepdims=True)
        gs_b = jnp.sum(oh * gs, axis=1, keepdims=True)
        wn = ge / (gs_b + 1e-16)
        ohw = oh * wn
        hg = lax.dot_general(ohw, h, (((0,), (0,)), ((), ())),
                             preferred_element_type=jnp.float32)
        o1 = lax.dot_general(hg, hw1_ref[...], (((1,), (1,)), ((), ())),
                             preferred_element_type=jnp.float32) + hb1_ref[...]
        o1 = jnp.maximum(o1, 0.0)
        o_ref[...] = (jnp.sum(o1 * hw2_ref[...], axis=1, keepdims=True)
                      + hb2_ref[0, 0])

    full = lambda *s: pl.BlockSpec(s, lambda: tuple(0 for _ in s))
    return pl.pallas_call(
        body,
        grid=(),
        in_specs=[
            full(N, C), full(N, 1),
            full(C // 2, C), full(1, C // 2), full(1, C // 2), full(1, 1),
            full(C, C), full(1, C), full(1, C), full(1, 1),
        ],
        out_specs=full(NG, 1),
        out_shape=_mk_f32((NG, 1)),
    )(h, batch2d, gw1, gb1, gw2, gb2, hw1, hb1, hw2, hb2)


def _sc_segsum(m, gidx2d, sidx2d):
    mesh = plsc.VectorSubcoreMesh(core_axis_name="c", subcore_axis_name="s")

    @functools.partial(
        pl.kernel,
        out_type=_mk_f32((R, C)),
        mesh=mesh,
        scratch_types=[
            pltpu.VMEM_SHARED((HR + NSUB, C), jnp.float32),
            pltpu.VMEM((GRP, 64), jnp.int32),
            pltpu.VMEM((GRP, 64), jnp.int32),
            pltpu.VMEM((64, C), jnp.float32),
            pltpu.SemaphoreType.DMA,
        ],
    )
    def k(m_hbm, g_hbm, s_hbm, out_hbm, acc, gvc, svc, rows, sem):
        c = lax.axis_index("c")
        s = lax.axis_index("s")
        base = c * HR
        trash = HR + lax.iota(jnp.int32, 16)

        @pl.loop(0, 64)
        def _zero_rows(r):
            for q in range(C // 16):
                rows[r, pl.ds(q * 16, 16)] = jnp.zeros((16,), jnp.float32)

        @pl.loop(0, RPT // 64)
        def _zero_acc(i):
            pltpu.sync_copy(rows, acc.at[pl.ds(s * RPT + i * 64, 64)])

        pltpu.sync_copy(rows.at[pl.ds(0, RPT % 64)],
                        acc.at[pl.ds(s * RPT + (RPT // 64) * 64, RPT % 64)])
        plsc.subcore_barrier()

        @pl.loop(0, PTR // GRP)
        def _group(g):
            row0 = s * PTR + g * GRP
            pltpu.sync_copy(g_hbm.at[pl.ds(row0, GRP)], gvc)
            pltpu.sync_copy(s_hbm.at[pl.ds(row0, GRP)], svc)
            for j in range(GRP):
                for q in range(4):
                    v = svc[j, pl.ds(q * 16, 16)] - base
                    ok = (v >= 0) & (v < HR)
                    svc[j, pl.ds(q * 16, 16)] = jnp.where(ok, v, trash)
            for j in range(GRP):
                pltpu.async_copy(m_hbm.at[gvc.at[j]], rows, sem).wait()
                pltpu.sync_copy(rows, acc.at[svc.at[j]], add=True)

        plsc.subcore_barrier()
        pltpu.sync_copy(acc.at[pl.ds(s * RPT, RPT)],
                        out_hbm.at[pl.ds(c * HR + s * RPT, RPT)])

    return k(m, gidx2d, sidx2d)


def kernel(x_small, params, x_type, x_tok, edge_index, edge_type, batch):
    x_type2d = x_type.astype(jnp.int32).reshape(N, 1)
    x_tok2d = x_tok.astype(jnp.int32).reshape(N, 1)
    src2d = edge_index[0].astype(jnp.int32).reshape(2500, 128)
    dst2d = edge_index[1].astype(jnp.int32).reshape(2500, 128)
    et2d = edge_type.astype(jnp.int32).reshape(2500, 128)
    batch2d = batch.astype(jnp.int32).reshape(N, 1)

    gidx, sidx = _edge_indices(src2d, dst2d, et2d)
    pad_rows = EIR - 5000
    gidx2d = jnp.concatenate(
        [gidx.reshape(5000, 64), jnp.zeros((pad_rows, 64), jnp.int32)], axis=0)
    sidx2d = jnp.concatenate(
        [sidx.reshape(5000, 64), jnp.full((pad_rows, 64), TRASH, jnp.int32)],
        axis=0)

    h = _build_h(x_type2d, x_tok2d, x_small)

    for blk in params['blocks']:
        convs = blk['convs']
        wih3 = jnp.stack([cv['W_ih'] for cv in convs])
        whh3 = jnp.stack([cv['W_hh'] for cv in convs])
        bih3 = jnp.stack([cv['b_ih'] for cv in convs]).reshape(NT, 1, 3 * C)
        bhh3 = jnp.stack([cv['b_hh'] for cv in convs]).reshape(NT, 1, 3 * C)
        xs = None
        for step in range(STEPS):
            w3 = jnp.stack([cv['weight'][step] for cv in convs])
            x_is_h = xs is None
            m2 = _mm3(h if x_is_h else xs, w3, x_is_h)
            msg2 = _sc_segsum(m2, gidx2d, sidx2d)
            xs = _gru3(msg2, h if x_is_h else xs,
                       wih3, whh3, bih3, bhh3, x_is_h)
        h = _block_end(h, xs, blk['ln_g'].reshape(1, C),
                       blk['ln_b'].reshape(1, C))

    out2d = _pool_head(
        h, batch2d,
        params['gate_W1'], params['gate_b1'].reshape(1, C // 2),
        params['gate_W2'], params['gate_b2'].reshape(1, 1),
        params['head_W1'], params['head_b1'].reshape(1, C),
        params['head_W2'], params['head_b2'].reshape(1, 1),
    )
    return out2d.reshape(NG)


# speedup vs baseline: 1.0946x; 1.0946x over previous
import functools

import jax
import jax.numpy as jnp
from jax import lax
from jax.experimental import pallas as pl
from jax.experimental.pallas import tpu as pltpu
from jax.experimental.pallas import tpu_sc as plsc

N = 10000
E = 320000
C = 128
STEPS = 3
NT = 3
NG = 16
LEAKY = 0.1

MR = 30208
HRO = 15000
AR = 15104
TRASH = 3 * N
EKR = 5000
PKR = 5376
NSUB = 16
GRP = 8
RPT = AR // NSUB
BM = 1000
NB = N // BM


def _mk_f32(shape):
    return jax.ShapeDtypeStruct(shape, jnp.float32)


def _bf(x):
    return x.astype(jnp.bfloat16)


def _build_h(x_type2d, x_tok2d, x_small):
    def body(xt_ref, xk_ref, xs_ref, h_ref):
        col = lax.broadcasted_iota(jnp.int32, (BM, C), 1)
        xt = xt_ref[...]
        xk = jnp.clip(xk_ref[...], 0, 61)
        xs = xs_ref[...]
        one = jnp.float32(1.0)
        zero = jnp.float32(0.0)
        h = jnp.where(col == xt, one, zero)
        h = h + jnp.where(col - 64 == xk, one, zero)
        h = h + jnp.where(col == 126, xs[:, 0:1], zero)
        h = h + jnp.where(col == 127, xs[:, 1:2], zero)
        h_ref[...] = h

    return pl.pallas_call(
        body,
        grid=(NB,),
        in_specs=[
            pl.BlockSpec((BM, 1), lambda i: (i, 0)),
            pl.BlockSpec((BM, 1), lambda i: (i, 0)),
            pl.BlockSpec((BM, 2), lambda i: (i, 0)),
        ],
        out_specs=pl.BlockSpec((BM, C), lambda i: (i, 0)),
        out_shape=_mk_f32((N, C)),
    )(x_type2d, x_tok2d, x_small)


def _edge_keys(src2d, dst2d, et2d):
    def body(s_ref, d_ref, t_ref, k_ref):
        t = t_ref[...]
        k_ref[...] = ((t * N + d_ref[...]) << 15) | (t * N + s_ref[...])

    blk = pl.BlockSpec((2500, 128), lambda: (0, 0))
    return pl.pallas_call(
        body,
        grid=(),
        in_specs=[blk, blk, blk],
        out_specs=blk,
        out_shape=jax.ShapeDtypeStruct((2500, 128), jnp.int32),
    )(src2d, dst2d, et2d)


def _mm3(x, w3, x_is_h):
    def body(x_ref, w_ref, o_ref):
        o_ref[...] = lax.dot_general(
            _bf(x_ref[0]), _bf(w_ref[0]), (((1,), (0,)), ((), ())),
            preferred_element_type=jnp.float32)

    if x_is_h:
        x = x.reshape(1, N, C)
        x_map = lambda t, i: (0, i, 0)
    else:
        x_map = lambda t, i: (t, i, 0)
    return pl.pallas_call(
        body,
        grid=(NT, NB),
        in_specs=[
            pl.BlockSpec((1, BM, C), x_map),
            pl.BlockSpec((1, C, C), lambda t, i: (t, 0, 0)),
        ],
        out_specs=pl.BlockSpec((BM, C), lambda t, i: (t * NB + i, 0)),
        out_shape=_mk_f32((MR, C)),
    )(x, w3)


def _gru3(msg, x, wih3, whh3, bih3, bhh3, x_is_h):
    def body(m_ref, x_ref, wi_ref, wh_ref, bi_ref, bh_ref, o_ref):
        m = m_ref[0]
        h = x_ref[0]
        gi = lax.dot_general(_bf(m), _bf(wi_ref[0]), (((1,), (1,)), ((), ())),
                             preferred_element_type=jnp.float32) + bi_ref[0]
        gh = lax.dot_general(_bf(h), _bf(wh_ref[0]), (((1,), (1,)), ((), ())),
                             preferred_element_type=jnp.float32) + bh_ref[0]
        r = jax.nn.sigmoid(gi[:, :C] + gh[:, :C])
        z = jax.nn.sigmoid(gi[:, C:2 * C] + gh[:, C:2 * C])
        n = jnp.tanh(gi[:, 2 * C:] + r * gh[:, 2 * C:])
        o_ref[0] = (1.0 - z) * n + z * h

    if x_is_h:
        x = x.reshape(1, N, C)
        x_map = lambda t, i: (0, i, 0)
    else:
        x_map = lambda t, i: (t, i, 0)
    def msg_map(t, i):
        gb = t * NB + i
        sec = jnp.where(gb >= 15, 1, 0)
        return (sec, gb - 15 * sec, 0)

    return pl.pallas_call(
        body,
        grid=(NT, NB),
        in_specs=[
            pl.BlockSpec((1, BM, C), msg_map),
            pl.BlockSpec((1, BM, C), x_map),
            pl.BlockSpec((1, 3 * C, C), lambda t, i: (t, 0, 0)),
            pl.BlockSpec((1, 3 * C, C), lambda t, i: (t, 0, 0)),
            pl.BlockSpec((1, 1, 3 * C), lambda t, i: (t, 0, 0)),
            pl.BlockSpec((1, 1, 3 * C), lambda t, i: (t, 0, 0)),
        ],
        out_specs=pl.BlockSpec((1, BM, C), lambda t, i: (t, i, 0)),
        out_shape=_mk_f32((NT, N, C)),
    )(msg, x, wih3, whh3, bih3, bhh3)


def _block_end(h, xs, ln_g, ln_b):
    def body(h_ref, xs_ref, g_ref, b_ref, o_ref):
        tot = h_ref[...] + xs_ref[0] + xs_ref[1] + xs_ref[2]
        mu = jnp.mean(tot, axis=-1, keepdims=True)
        d = tot - mu
        var = jnp.mean(d * d, axis=-1, keepdims=True)
        y = d * lax.rsqrt(var + 1e-5) * g_ref[...] + b_ref[...]
        o_ref[...] = jnp.where(y >= 0, y, LEAKY * y)

    return pl.pallas_call(
        body,
        grid=(NB,),
        in_specs=[
            pl.BlockSpec((BM, C), lambda i: (i, 0)),
            pl.BlockSpec((3, BM, C), lambda i: (0, i, 0)),
            pl.BlockSpec((1, C), lambda i: (0, 0)),
            pl.BlockSpec((1, C), lambda i: (0, 0)),
        ],
        out_specs=pl.BlockSpec((BM, C), lambda i: (i, 0)),
        out_shape=_mk_f32((N, C)),
    )(h, xs, ln_g, ln_b)


def _pool_head(h, batch2d, gw1, gb1, gw2, gb2, hw1, hb1, hw2, hb2):
    def body(h_ref, b_ref, gw1_ref, gb1_ref, gw2_ref, gb2_ref,
             hw1_ref, hb1_ref, hw2_ref, hb2_ref, o_ref):
        h = h_ref[...]
        g1 = lax.dot_general(_bf(h), _bf(gw1_ref[...]), (((1,), (1,)), ((), ())),
                             preferred_element_type=jnp.float32) + gb1_ref[...]
        g1 = jnp.where(g1 >= 0, g1, LEAKY * g1)
        g = jnp.sum(_bf(g1).astype(jnp.float32) * _bf(gw2_ref[...]).astype(jnp.float32), axis=1, keepdims=True) + gb2_ref[0, 0]
        b = b_ref[...]
        oh = (b == lax.broadcasted_iota(jnp.int32, (N, NG), 1)).astype(jnp.float32)
        neg = jnp.float32(-3e38)
        gmax = jnp.max(jnp.where(oh > 0, g, neg), axis=0, keepdims=True)
        gmax = jnp.where(gmax > -1e38, gmax, 0.0)
        gmax_b = jnp.sum(oh * gmax, axis=1, keepdims=True)
        ge = jnp.exp(g - gmax_b)
        gs = jnp.sum(oh * ge, axis=0, keepdims=True)
        gs_b = jnp.sum(oh * gs, axis=1, keepdims=True)
        wn = ge / (gs_b + 1e-16)
        wh = h * wn
        hg = jnp.concatenate(
            [jnp.sum(jnp.where(b == gi, wh, 0.0), axis=0, keepdims=True)
             for gi in range(NG)], axis=0)
        o1 = lax.dot_general(_bf(hg), _bf(hw1_ref[...]), (((1,), (1,)), ((), ())),
                             preferred_element_type=jnp.float32) + hb1_ref[...]
        o1 = jnp.maximum(o1, 0.0)
        o_ref[...] = (jnp.sum(_bf(o1).astype(jnp.float32) * _bf(hw2_ref[...]).astype(jnp.float32), axis=1, keepdims=True)
                      + hb2_ref[0, 0])

    full = lambda *s: pl.BlockSpec(s, lambda: tuple(0 for _ in s))
    return pl.pallas_call(
        body,
        grid=(),
        in_specs=[
            full(N, C), full(N, 1),
            full(C // 2, C), full(1, C // 2), full(1, C // 2), full(1, 1),
            full(C, C), full(1, C), full(1, C), full(1, 1),
        ],
        out_specs=full(NG, 1),
        out_shape=_mk_f32((NG, 1)),
    )(h, batch2d, gw1, gb1, gw2, gb2, hw1, hb1, hw2, hb2)


def _sc_segsum(m, pk2d, scal):
    mesh = plsc.VectorSubcoreMesh(core_axis_name="c", subcore_axis_name="s")

    @functools.partial(
        pl.kernel,
        out_type=_mk_f32((2, AR, C)),
        mesh=mesh,
        scratch_types=[
            pltpu.VMEM_SHARED((AR, C), jnp.float32),
            pltpu.VMEM((GRP, 64), jnp.int32),
            pltpu.VMEM((1, 64), jnp.int32),
            pltpu.VMEM((1, 64), jnp.int32),
            pltpu.VMEM((64, C), jnp.float32),
            pltpu.VMEM((16,), jnp.int32),
            pltpu.SemaphoreType.DMA,
        ],
    )
    def k(m_hbm, pk_hbm, scal_hbm, out_hbm, acc, ib, ig, isv, rows,
          sm, sem):
        c = lax.axis_index("c")
        s = lax.axis_index("s")
        base = c * HRO
        trash = HRO + lax.iota(jnp.int32, 16)
        pltpu.sync_copy(scal_hbm, sm)

        @pl.loop(0, 64)
        def _zero_rows(r):
            for q in range(C // 16):
                rows[r, pl.ds(q * 16, 16)] = jnp.zeros((16,), jnp.float32)

        @pl.loop(0, RPT // 64)
        def _zero_acc(i):
            pltpu.sync_copy(rows, acc.at[pl.ds(s * RPT + i * 64, 64)])

        pltpu.sync_copy(rows.at[pl.ds(0, RPT % 64)],
                        acc.at[pl.ds(s * RPT + (RPT // 64) * 64, RPT % 64)])
        plsc.subcore_barrier()

        sv = sm[...]
        k0 = sv[0]
        r1 = sv[1]
        k1 = sv[2]
        nrows = jnp.where(c == 0, k0, k1)
        rowbase = jnp.where(c == 0, s * k0, r1 + s * k1)

        def _group(g, carry):
            row0 = pl.multiple_of(rowbase + g * GRP, GRP)
            pltpu.sync_copy(pk_hbm.at[pl.ds(row0, GRP)], ib)
            for j in range(GRP):
                for q in range(4):
                    v = ib[j, pl.ds(q * 16, 16)]
                    ig[0, pl.ds(q * 16, 16)] = v & 32767
                    d = (v >> 15) - base
                    ok = (d >= 0) & (d < HRO)
                    isv[0, pl.ds(q * 16, 16)] = jnp.where(ok, d, trash)
                pltpu.async_copy(m_hbm.at[ig.at[0]], rows, sem).wait()
                pltpu.sync_copy(rows, acc.at[isv.at[0]], add=True)
            return carry

        lax.fori_loop(0, nrows // GRP, _group, 0)

        plsc.subcore_barrier()
        pltpu.sync_copy(acc.at[pl.ds(s * RPT, RPT)],
                        out_hbm.at[c].at[pl.ds(s * RPT, RPT)])

    return k(m, pk2d, scal)


def kernel(x_small, params, x_type, x_tok, edge_index, edge_type, batch):
    x_type2d = x_type.astype(jnp.int32).reshape(N, 1)
    x_tok2d = x_tok.astype(jnp.int32).reshape(N, 1)
    src2d = edge_index[0].astype(jnp.int32).reshape(2500, 128)
    dst2d = edge_index[1].astype(jnp.int32).reshape(2500, 128)
    et2d = edge_type.astype(jnp.int32).reshape(2500, 128)
    batch2d = batch.astype(jnp.int32).reshape(N, 1)

    key = _edge_keys(src2d, dst2d, et2d)
    skey = jnp.sort(key.reshape(E))
    pk2d = jnp.concatenate(
        [skey, jnp.full((PKR * 64 - E,), TRASH << 15, jnp.int32)]
    ).reshape(PKR, 64)
    c0 = jnp.sum((skey < (HRO << 15)).astype(jnp.int32))
    r0e = (c0 + 63) // 64
    k0 = ((r0e + 127) // 128) * 8
    r1 = ((c0 // 64) // 8) * 8
    k1 = ((EKR + 120 - r1 + 127) // 128) * 8
    scal = jnp.stack([k0, r1, k1] + [k0] * 13).astype(jnp.int32)

    h = _build_h(x_type2d, x_tok2d, x_small)

    for blk in params['blocks']:
        convs = blk['convs']
        wih3 = jnp.stack([cv['W_ih'] for cv in convs])
        whh3 = jnp.stack([cv['W_hh'] for cv in convs])
        bih3 = jnp.stack([cv['b_ih'] for cv in convs]).reshape(NT, 1, 3 * C)
        bhh3 = jnp.stack([cv['b_hh'] for cv in convs]).reshape(NT, 1, 3 * C)
        xs = None
        for step in range(STEPS):
            w3 = jnp.stack([cv['weight'][step] for cv in convs])
            x_is_h = xs is None
            m2 = _mm3(h if x_is_h else xs, w3, x_is_h)
            msg2 = _sc_segsum(m2, pk2d, scal)
            xs = _gru3(msg2, h if x_is_h else xs,
                       wih3, whh3, bih3, bhh3, x_is_h)
        h = _block_end(h, xs, blk['ln_g'].reshape(1, C),
                       blk['ln_b'].reshape(1, C))

    out2d = _pool_head(
        h, batch2d,
        params['gate_W1'], params['gate_b1'].reshape(1, C // 2),
        params['gate_W2'], params['gate_b2'].reshape(1, 1),
        params['head_W1'], params['head_b1'].reshape(1, C),
        params['head_W2'], params['head_b2'].reshape(1, 1),
    )
    return out2d.reshape(NG)
